# Initial kernel scaffold; baseline (speedup 1.0000x reference)
import jax
import jax.numpy as jnp
from jax.experimental import pallas as pl


def kernel(z, batch, pos, edges, emb, iw1, ib1, iw2, ib2, cw1, cw2, cb2, lw, lb, lin1_w, lin1_b, m1w, m1b, m2w, m2b):
    raise NotImplementedError("write your pallas kernel here")


# speedup vs baseline: 11.5928x; 11.5928x over previous
import functools

import jax
import jax.numpy as jnp
from jax import lax
from jax.experimental import pallas as pl
from jax.experimental.pallas import tpu as pltpu
from jax.experimental.pallas import tpu_sc as plsc

_CUTOFF = 10.0
_HID = 128
_NG = 50
_NGP = 64
_R = 32
_CT = 128
_NP = 10240
_NB = _NP // _R
_P = _R * _CT


def _ssp(x):
    return jnp.maximum(x, 0.0) + jnp.log1p(jnp.exp(-jnp.abs(x))) - jnp.log(2.0)


def _sc_embed(emb, zp):
    info = plsc.get_sparse_core_info()
    nc, ns = info.num_cores, info.num_subcores
    nw = nc * ns
    b_per_w = _NP // nw
    d = emb.shape[1]
    mesh = plsc.VectorSubcoreMesh(core_axis_name="c", subcore_axis_name="s")

    @functools.partial(
        pl.kernel,
        mesh=mesh,
        out_type=jax.ShapeDtypeStruct((_NP, d), jnp.float32),
        scratch_types=[
            pltpu.VMEM((b_per_w,), jnp.int32),
            pltpu.VMEM((b_per_w, d), jnp.float32),
            pltpu.SemaphoreType.DMA,
        ],
    )
    def gather_kernel(table_hbm, idx_hbm, out_hbm, idx_v, rows_v, sem):
        wid = lax.axis_index("s") * nc + lax.axis_index("c")
        base = wid * b_per_w
        pltpu.sync_copy(idx_hbm.at[pl.ds(base, b_per_w)], idx_v)
        pltpu.async_copy(table_hbm.at[idx_v], rows_v, sem).wait()
        pltpu.sync_copy(rows_v, out_hbm.at[pl.ds(base, b_per_w)])

    return gather_kernel(emb, zp)


def _banded_body(tlo_ref, tcnt_ref, nodef_ref, xj_ref, iw1_ref, ib1_ref,
                 iw2_ref, ib2_ref, out_ref):
    b = pl.program_id(0)
    tlo = tlo_ref[b]
    tcnt = tcnt_ref[b]
    r0 = b * _R

    f32 = jnp.float32
    step = _CUTOFF / (_NG - 1)
    coeff = -0.5 / (step * step)
    gi = lax.broadcasted_iota(jnp.int32, (1, _NGP), 1).astype(f32)
    off = jnp.where(gi < _NG, gi * step, 1e6)

    pidx = lax.broadcasted_iota(jnp.int32, (_P, 1), 0)
    p_div = pidx // _CT
    p_mod = pidx - p_div * _CT

    rowdat = nodef_ref[pl.ds(r0, _R), :]
    row_f = jnp.repeat(rowdat, _CT, axis=0)
    br = row_f[:, 0:1]
    nr = row_f[:, 1:2]
    prx = row_f[:, 2:3]
    pry = row_f[:, 3:4]
    prz = row_f[:, 4:5]
    ridx = p_div + r0

    iw1v = iw1_ref[...]
    ib1v = ib1_ref[...]
    iw2v = iw2_ref[...]
    ib2v = ib2_ref[...]

    def tile_body(t, acc):
        c0 = (tlo + t) * _CT
        coldat = nodef_ref[pl.ds(c0, _CT), :]
        xjc = xj_ref[pl.ds(c0, _CT), :]
        col_f = jnp.tile(coldat, (_R, 1))
        bc = col_f[:, 0:1]
        nc = col_f[:, 1:2]
        pcx = col_f[:, 2:3]
        pcy = col_f[:, 3:4]
        pcz = col_f[:, 4:5]
        cidx = p_mod + c0

        dot3 = prx * pcx + pry * pcy + prz * pcz
        d2 = (nr + nc) - 2.0 * dot3
        m = (d2 < _CUTOFF * _CUTOFF) & (br == bc) & (ridx != cidx)
        dx = prx - pcx
        dy = pry - pcy
        dz = prz - pcz
        ew = jnp.sqrt(dx * dx + dy * dy + dz * dz)
        cw = 0.5 * (jnp.cos(ew * (jnp.pi / _CUTOFF)) + 1.0)
        cm = jnp.where(m, cw, 0.0)

        ea = jnp.exp(coeff * (ew - off) ** 2)
        t1 = _ssp(jnp.dot(ea, iw1v, preferred_element_type=f32) + ib1v)
        w = jnp.dot(t1, iw2v, preferred_element_type=f32) + ib2v
        v = w * cm
        v3 = v.reshape(_R, _CT, _HID)
        return acc + jnp.sum(v3 * xjc[None, :, :], axis=1)

    acc = lax.fori_loop(0, tcnt, tile_body, jnp.zeros((_R, _HID), f32))
    out_ref[...] = acc


def _banded(tlo, tcnt, nodef, xj, iw1p, ib1, iw2, ib2):
    grid_spec = pltpu.PrefetchScalarGridSpec(
        num_scalar_prefetch=2,
        grid=(_NB,),
        in_specs=[
            pl.BlockSpec((_NP, 8), lambda b, *_: (0, 0)),
            pl.BlockSpec((_NP, _HID), lambda b, *_: (0, 0)),
            pl.BlockSpec((_NGP, _HID), lambda b, *_: (0, 0)),
            pl.BlockSpec((1, _HID), lambda b, *_: (0, 0)),
            pl.BlockSpec((_HID, _HID), lambda b, *_: (0, 0)),
            pl.BlockSpec((1, _HID), lambda b, *_: (0, 0)),
        ],
        out_specs=pl.BlockSpec((_R, _HID), lambda b, *_: (b, 0)),
    )
    return pl.pallas_call(
        _banded_body,
        grid_spec=grid_spec,
        out_shape=jax.ShapeDtypeStruct((_NP, _HID), jnp.float32),
    )(tlo, tcnt, nodef, xj, iw1p, ib1.reshape(1, _HID), iw2,
      ib2.reshape(1, _HID))


def _mm_body(x_ref, w_ref, o_ref):
    o_ref[...] = jnp.dot(x_ref[...], w_ref[...],
                         preferred_element_type=jnp.float32)


def _mm(x, w):
    m, k = x.shape
    n = w.shape[1]
    blk = min(1024, m)
    return pl.pallas_call(
        _mm_body,
        grid=(m // blk,),
        in_specs=[pl.BlockSpec((blk, k), lambda i: (i, 0)),
                  pl.BlockSpec((k, n), lambda i: (0, 0))],
        out_specs=pl.BlockSpec((blk, n), lambda i: (i, 0)),
        out_shape=jax.ShapeDtypeStruct((m, n), jnp.float32),
    )(x, w)


def _update_body(h_ref, agg_ref, cw2_ref, cb2_ref, lw_ref, lb_ref, o_ref):
    t = _ssp(jnp.dot(agg_ref[...], cw2_ref[...],
                     preferred_element_type=jnp.float32) + cb2_ref[...])
    o_ref[...] = h_ref[...] + jnp.dot(
        t, lw_ref[...], preferred_element_type=jnp.float32) + lb_ref[...]


def _update(h, agg, cw2, cb2, lw, lb):
    blk = min(1024, _NP)
    return pl.pallas_call(
        _update_body,
        grid=(_NP // blk,),
        in_specs=[pl.BlockSpec((blk, _HID), lambda i: (i, 0)),
                  pl.BlockSpec((blk, _HID), lambda i: (i, 0)),
                  pl.BlockSpec((_HID, _HID), lambda i: (0, 0)),
                  pl.BlockSpec((1, _HID), lambda i: (0, 0)),
                  pl.BlockSpec((_HID, _HID), lambda i: (0, 0)),
                  pl.BlockSpec((1, _HID), lambda i: (0, 0))],
        out_specs=pl.BlockSpec((blk, _HID), lambda i: (i, 0)),
        out_shape=jax.ShapeDtypeStruct((_NP, _HID), jnp.float32),
    )(h, agg, cw2, cb2.reshape(1, _HID), lw, lb.reshape(1, _HID))


def _lin1_body(h_ref, w_ref, b_ref, o_ref):
    o_ref[...] = jnp.dot(h_ref[...], w_ref[...],
                         preferred_element_type=jnp.float32) + b_ref[...]


def _lin1(h, w, b):
    blk = min(1024, _NP)
    n = w.shape[1]
    return pl.pallas_call(
        _lin1_body,
        grid=(_NP // blk,),
        in_specs=[pl.BlockSpec((blk, _HID), lambda i: (i, 0)),
                  pl.BlockSpec((_HID, n), lambda i: (0, 0)),
                  pl.BlockSpec((1, n), lambda i: (0, 0))],
        out_specs=pl.BlockSpec((blk, n), lambda i: (i, 0)),
        out_shape=jax.ShapeDtypeStruct((_NP, n), jnp.float32),
    )(h, w, b.reshape(1, n))


def _readout_body(p_ref, m1w_ref, m1b_ref, m2w_ref, m2b_ref, o_ref):
    t = jax.nn.relu(jnp.dot(p_ref[...], m1w_ref[...],
                            preferred_element_type=jnp.float32) + m1b_ref[...])
    o_ref[...] = jnp.dot(t, m2w_ref[...],
                         preferred_element_type=jnp.float32) + m2b_ref[...]


def _readout(pairp, m1w, m1b, m2w, m2b):
    mp = pairp.shape[0]
    blk = 512
    return pl.pallas_call(
        _readout_body,
        grid=(mp // blk,),
        in_specs=[pl.BlockSpec((blk, _HID), lambda i: (i, 0)),
                  pl.BlockSpec((_HID, _HID), lambda i: (0, 0)),
                  pl.BlockSpec((1, _HID), lambda i: (0, 0)),
                  pl.BlockSpec((_HID, 1), lambda i: (0, 0)),
                  pl.BlockSpec((1, 1), lambda i: (0, 0))],
        out_specs=pl.BlockSpec((blk, 1), lambda i: (i, 0)),
        out_shape=jax.ShapeDtypeStruct((mp, 1), jnp.float32),
    )(pairp, m1w, m1b.reshape(1, _HID), m2w, m2b.reshape(1, 1))


def kernel(z, batch, pos, edges, emb, iw1, ib1, iw2, ib2, cw1, cw2, cb2,
           lw, lb, lin1_w, lin1_b, m1w, m1b, m2w, m2b):
    n = pos.shape[0]
    flat = edges[0].reshape(-1)
    pos_s = jnp.take(pos, flat, axis=0).astype(jnp.float32)
    nrm = (pos_s * pos_s).sum(1)
    batch_i = batch.astype(jnp.int32)

    padn = _NP - n
    batch_f = jnp.pad(batch_i, (0, padn),
                      constant_values=2 ** 24 - 1).astype(jnp.float32)
    nrm_p = jnp.pad(nrm, (0, padn))
    pos_p = jnp.pad(pos_s, ((0, padn), (0, 0)))
    gidx = jnp.arange(_NP, dtype=jnp.float32)
    zeros = jnp.zeros((_NP,), jnp.float32)
    nodef = jnp.stack([batch_f, nrm_p, pos_p[:, 0], pos_p[:, 1],
                       pos_p[:, 2], gidx, zeros, zeros], axis=1)

    row0 = jnp.arange(_NB, dtype=jnp.int32) * _R
    rlast = jnp.minimum(row0 + _R - 1, n - 1)
    bfirst = batch_i[jnp.minimum(row0, n - 1)]
    cs = jnp.searchsorted(batch_i, bfirst, side="left").astype(jnp.int32)
    ce = jnp.searchsorted(batch_i, batch_i[rlast], side="right").astype(jnp.int32)
    tlo = cs // _CT
    thi = (ce + _CT - 1) // _CT
    tcnt = jnp.where(row0 < n, thi - tlo, 0).astype(jnp.int32)

    iw1p = jnp.pad(iw1, ((0, 0), (0, _NGP - _NG), (0, 0)))

    zp = jnp.pad(z.astype(jnp.int32), (0, padn))
    h = _sc_embed(emb.astype(jnp.float32), zp)

    for i in range(6):
        xj = _mm(h, cw1[i])
        agg = _banded(tlo, tcnt, nodef, xj, iw1p[i], ib1[i], iw2[i], ib2[i])
        h = _update(h, agg, cw2[i], cb2[i], lw[i], lb[i])

    ne = _lin1(h, lin1_w, lin1_b)
    pair = ne[:n].reshape(n // 2, 2 * ne.shape[1])
    mp = 5120
    pairp = jnp.pad(pair, ((0, mp - n // 2), (0, 0)))
    outp = _readout(pairp, m1w, m1b, m2w, m2b)
    return outp[: n // 2, 0]


# speedup vs baseline: 26.7294x; 2.3057x over previous
import functools

import jax
import jax.numpy as jnp
from jax import lax
from jax.experimental import pallas as pl
from jax.experimental.pallas import tpu as pltpu
from jax.experimental.pallas import tpu_sc as plsc

_CUTOFF = 10.0
_HID = 128
_NG = 50
_NGP = 64
_R = 32
_CT = 128
_NP = 10240
_NB = _NP // _R
_P = _R * _CT


def _ssp(x):
    return jnp.maximum(x, 0.0) + jnp.log1p(jnp.exp(-jnp.abs(x))) - jnp.log(2.0)


def _sc_embed(emb, zp):
    info = plsc.get_sparse_core_info()
    nc, ns = info.num_cores, info.num_subcores
    nw = nc * ns
    b_per_w = _NP // nw
    d = emb.shape[1]
    mesh = plsc.VectorSubcoreMesh(core_axis_name="c", subcore_axis_name="s")

    @functools.partial(
        pl.kernel,
        mesh=mesh,
        out_type=jax.ShapeDtypeStruct((_NP, d), jnp.float32),
        scratch_types=[
            pltpu.VMEM((b_per_w,), jnp.int32),
            pltpu.VMEM((b_per_w, d), jnp.float32),
            pltpu.SemaphoreType.DMA,
        ],
    )
    def gather_kernel(table_hbm, idx_hbm, out_hbm, idx_v, rows_v, sem):
        wid = lax.axis_index("s") * nc + lax.axis_index("c")
        base = wid * b_per_w
        pltpu.sync_copy(idx_hbm.at[pl.ds(base, b_per_w)], idx_v)
        pltpu.async_copy(table_hbm.at[idx_v], rows_v, sem).wait()
        pltpu.sync_copy(rows_v, out_hbm.at[pl.ds(base, b_per_w)])

    return gather_kernel(emb, zp)


def _banded_body(tlo_ref, tcnt_ref, nodef_ref, nodet_ref, xj_ref, iw1_ref,
                 ib1_ref, iw2_ref, ib2_ref, out_ref):
    b = pl.program_id(0)
    tlo = tlo_ref[b]
    tcnt = tcnt_ref[b]
    r0 = b * _R

    f32 = jnp.float32
    step = _CUTOFF / (_NG - 1)
    coeff = -0.5 / (step * step)
    gi = lax.broadcasted_iota(jnp.int32, (1, _NGP), 1).astype(f32)
    off = jnp.where(gi < _NG, gi * step, 1e6)

    rowdat = nodef_ref[pl.ds(r0, _R), :]
    br = rowdat[:, 0:1]
    nr = rowdat[:, 1:2]
    prx = rowdat[:, 2:3]
    pry = rowdat[:, 3:4]
    prz = rowdat[:, 4:5]
    ridx = r0 + lax.broadcasted_iota(jnp.int32, (_R, 1), 0)

    iw1v = iw1_ref[...]
    ib1v = ib1_ref[...]
    iw2v = iw2_ref[...]
    ib2v = ib2_ref[...]

    pi_r = lax.broadcasted_iota(jnp.int32, (_P, _R), 0)
    ri = lax.broadcasted_iota(jnp.int32, (_P, _R), 1)
    sel_r = (pi_r // _CT == ri).astype(f32)
    pi_c = lax.broadcasted_iota(jnp.int32, (_P, _CT), 0)
    ci = lax.broadcasted_iota(jnp.int32, (_P, _CT), 1)
    sel_c = (pi_c % _CT == ci).astype(f32)

    def _flatten2(a2, b2):
        ab = jnp.concatenate([a2, b2], axis=1)
        ex = jnp.dot(sel_r, ab, preferred_element_type=f32,
                     precision=lax.Precision.HIGHEST)
        af = jnp.sum(ex[:, :_CT] * sel_c, axis=1, keepdims=True)
        bf = jnp.sum(ex[:, _CT:] * sel_c, axis=1, keepdims=True)
        return af, bf

    def tile_body(t, acc):
        tt = tlo + t
        c0 = tt * _CT
        colt = nodet_ref[pl.ds(tt, 1), :, :].reshape(8, _CT)
        xjc = xj_ref[pl.ds(c0, _CT), :]
        bc = colt[0:1, :]
        nc = colt[1:2, :]
        pcx = colt[2:3, :]
        pcy = colt[3:4, :]
        pcz = colt[4:5, :]
        cidx = c0 + lax.broadcasted_iota(jnp.int32, (1, _CT), 1)

        dot3 = prx * pcx + pry * pcy + prz * pcz
        d2 = (nr + nc) - 2.0 * dot3
        m = (d2 < _CUTOFF * _CUTOFF) & (br == bc) & (ridx != cidx)
        dx = prx - pcx
        dy = pry - pcy
        dz = prz - pcz
        ew = jnp.sqrt(dx * dx + dy * dy + dz * dz)
        cw = 0.5 * (jnp.cos(ew * (jnp.pi / _CUTOFF)) + 1.0)
        cm2 = jnp.where(m, cw, 0.0)

        ew_f, cm_f = _flatten2(ew, cm2)
        ea = jnp.exp(coeff * (ew_f - off) ** 2)
        t1 = _ssp(jnp.dot(ea, iw1v, preferred_element_type=f32) + ib1v)
        w = jnp.dot(t1, iw2v, preferred_element_type=f32) + ib2v
        v = w * cm_f
        v3 = v.reshape(_R, _CT, _HID)
        return acc + jnp.sum(v3 * xjc[None, :, :], axis=1)

    acc = lax.fori_loop(0, tcnt, tile_body, jnp.zeros((_R, _HID), f32))
    out_ref[...] = acc


def _banded(tlo, tcnt, nodef, nodet, xj, iw1p, ib1, iw2, ib2):
    grid_spec = pltpu.PrefetchScalarGridSpec(
        num_scalar_prefetch=2,
        grid=(_NB,),
        in_specs=[
            pl.BlockSpec((_NP, 8), lambda b, *_: (0, 0)),
            pl.BlockSpec((_NP // _CT, 8, _CT), lambda b, *_: (0, 0, 0)),
            pl.BlockSpec((_NP, _HID), lambda b, *_: (0, 0)),
            pl.BlockSpec((_NGP, _HID), lambda b, *_: (0, 0)),
            pl.BlockSpec((1, _HID), lambda b, *_: (0, 0)),
            pl.BlockSpec((_HID, _HID), lambda b, *_: (0, 0)),
            pl.BlockSpec((1, _HID), lambda b, *_: (0, 0)),
        ],
        out_specs=pl.BlockSpec((_R, _HID), lambda b, *_: (b, 0)),
    )
    return pl.pallas_call(
        _banded_body,
        grid_spec=grid_spec,
        out_shape=jax.ShapeDtypeStruct((_NP, _HID), jnp.float32),
    )(tlo, tcnt, nodef, nodet, xj, iw1p, ib1.reshape(1, _HID), iw2,
      ib2.reshape(1, _HID))


def _mm_body(x_ref, w_ref, o_ref):
    o_ref[...] = jnp.dot(x_ref[...], w_ref[...],
                         preferred_element_type=jnp.float32)


def _mm(x, w):
    m, k = x.shape
    n = w.shape[1]
    blk = min(1024, m)
    return pl.pallas_call(
        _mm_body,
        grid=(m // blk,),
        in_specs=[pl.BlockSpec((blk, k), lambda i: (i, 0)),
                  pl.BlockSpec((k, n), lambda i: (0, 0))],
        out_specs=pl.BlockSpec((blk, n), lambda i: (i, 0)),
        out_shape=jax.ShapeDtypeStruct((m, n), jnp.float32),
    )(x, w)


def _update_body(h_ref, agg_ref, cw2_ref, cb2_ref, lw_ref, lb_ref, o_ref):
    t = _ssp(jnp.dot(agg_ref[...], cw2_ref[...],
                     preferred_element_type=jnp.float32) + cb2_ref[...])
    o_ref[...] = h_ref[...] + jnp.dot(
        t, lw_ref[...], preferred_element_type=jnp.float32) + lb_ref[...]


def _update(h, agg, cw2, cb2, lw, lb):
    blk = min(1024, _NP)
    return pl.pallas_call(
        _update_body,
        grid=(_NP // blk,),
        in_specs=[pl.BlockSpec((blk, _HID), lambda i: (i, 0)),
                  pl.BlockSpec((blk, _HID), lambda i: (i, 0)),
                  pl.BlockSpec((_HID, _HID), lambda i: (0, 0)),
                  pl.BlockSpec((1, _HID), lambda i: (0, 0)),
                  pl.BlockSpec((_HID, _HID), lambda i: (0, 0)),
                  pl.BlockSpec((1, _HID), lambda i: (0, 0))],
        out_specs=pl.BlockSpec((blk, _HID), lambda i: (i, 0)),
        out_shape=jax.ShapeDtypeStruct((_NP, _HID), jnp.float32),
    )(h, agg, cw2, cb2.reshape(1, _HID), lw, lb.reshape(1, _HID))


def _lin1_body(h_ref, w_ref, b_ref, o_ref):
    o_ref[...] = jnp.dot(h_ref[...], w_ref[...],
                         preferred_element_type=jnp.float32) + b_ref[...]


def _lin1(h, w, b):
    blk = min(1024, _NP)
    n = w.shape[1]
    return pl.pallas_call(
        _lin1_body,
        grid=(_NP // blk,),
        in_specs=[pl.BlockSpec((blk, _HID), lambda i: (i, 0)),
                  pl.BlockSpec((_HID, n), lambda i: (0, 0)),
                  pl.BlockSpec((1, n), lambda i: (0, 0))],
        out_specs=pl.BlockSpec((blk, n), lambda i: (i, 0)),
        out_shape=jax.ShapeDtypeStruct((_NP, n), jnp.float32),
    )(h, w, b.reshape(1, n))


def _readout_body(p_ref, m1w_ref, m1b_ref, m2w_ref, m2b_ref, o_ref):
    t = jax.nn.relu(jnp.dot(p_ref[...], m1w_ref[...],
                            preferred_element_type=jnp.float32) + m1b_ref[...])
    o_ref[...] = jnp.dot(t, m2w_ref[...],
                         preferred_element_type=jnp.float32) + m2b_ref[...]


def _readout(pairp, m1w, m1b, m2w, m2b):
    mp = pairp.shape[0]
    blk = 512
    return pl.pallas_call(
        _readout_body,
        grid=(mp // blk,),
        in_specs=[pl.BlockSpec((blk, _HID), lambda i: (i, 0)),
                  pl.BlockSpec((_HID, _HID), lambda i: (0, 0)),
                  pl.BlockSpec((1, _HID), lambda i: (0, 0)),
                  pl.BlockSpec((_HID, 1), lambda i: (0, 0)),
                  pl.BlockSpec((1, 1), lambda i: (0, 0))],
        out_specs=pl.BlockSpec((blk, 1), lambda i: (i, 0)),
        out_shape=jax.ShapeDtypeStruct((mp, 1), jnp.float32),
    )(pairp, m1w, m1b.reshape(1, _HID), m2w, m2b.reshape(1, 1))


def kernel(z, batch, pos, edges, emb, iw1, ib1, iw2, ib2, cw1, cw2, cb2,
           lw, lb, lin1_w, lin1_b, m1w, m1b, m2w, m2b):
    n = pos.shape[0]
    flat = edges[0].reshape(-1)
    pos_s = jnp.take(pos, flat, axis=0).astype(jnp.float32)
    nrm = (pos_s * pos_s).sum(1)
    batch_i = batch.astype(jnp.int32)

    padn = _NP - n
    batch_f = jnp.pad(batch_i, (0, padn),
                      constant_values=2 ** 24 - 1).astype(jnp.float32)
    nrm_p = jnp.pad(nrm, (0, padn))
    pos_p = jnp.pad(pos_s, ((0, padn), (0, 0)))
    gidx = jnp.arange(_NP, dtype=jnp.float32)
    zeros = jnp.zeros((_NP,), jnp.float32)
    nodef = jnp.stack([batch_f, nrm_p, pos_p[:, 0], pos_p[:, 1],
                       pos_p[:, 2], gidx, zeros, zeros], axis=1)
    nodet = nodef.T.reshape(8, _NP // _CT, _CT).transpose(1, 0, 2)

    row0 = jnp.arange(_NB, dtype=jnp.int32) * _R
    rlast = jnp.minimum(row0 + _R - 1, n - 1)
    bfirst = batch_i[jnp.minimum(row0, n - 1)]
    cs = jnp.searchsorted(batch_i, bfirst, side="left").astype(jnp.int32)
    ce = jnp.searchsorted(batch_i, batch_i[rlast], side="right").astype(jnp.int32)
    tlo = cs // _CT
    thi = (ce + _CT - 1) // _CT
    tcnt = jnp.where(row0 < n, thi - tlo, 0).astype(jnp.int32)

    iw1p = jnp.pad(iw1, ((0, 0), (0, _NGP - _NG), (0, 0)))

    zp = jnp.pad(z.astype(jnp.int32), (0, padn))
    h = _sc_embed(emb.astype(jnp.float32), zp)

    for i in range(6):
        xj = _mm(h, cw1[i])
        agg = _banded(tlo, tcnt, nodef, nodet, xj, iw1p[i], ib1[i], iw2[i],
                      ib2[i])
        h = _update(h, agg, cw2[i], cb2[i], lw[i], lb[i])

    ne = _lin1(h, lin1_w, lin1_b)
    pair = ne[:n].reshape(n // 2, 2 * ne.shape[1])
    mp = 5120
    pairp = jnp.pad(pair, ((0, mp - n // 2), (0, 0)))
    outp = _readout(pairp, m1w, m1b, m2w, m2b)
    return outp[: n // 2, 0]


# speedup vs baseline: 46.6427x; 1.7450x over previous
import functools

import jax
import jax.numpy as jnp
from jax import lax
from jax.experimental import pallas as pl
from jax.experimental.pallas import tpu as pltpu
from jax.experimental.pallas import tpu_sc as plsc

_CUTOFF = 10.0
_HID = 128
_NG = 50
_NGP = 64
_R = 32
_CT = 128
_NP = 10240
_NB = _NP // _R
_P = _R * _CT


def _ssp(x):
    return jnp.maximum(x, 0.0) + jnp.log1p(jnp.exp(-jnp.abs(x))) - jnp.log(2.0)


def _sc_embed(emb, zp):
    info = plsc.get_sparse_core_info()
    nc, ns = info.num_cores, info.num_subcores
    nw = nc * ns
    b_per_w = _NP // nw
    d = emb.shape[1]
    mesh = plsc.VectorSubcoreMesh(core_axis_name="c", subcore_axis_name="s")

    @functools.partial(
        pl.kernel,
        mesh=mesh,
        out_type=jax.ShapeDtypeStruct((_NP, d), jnp.float32),
        scratch_types=[
            pltpu.VMEM((b_per_w,), jnp.int32),
            pltpu.VMEM((b_per_w, d), jnp.float32),
            pltpu.SemaphoreType.DMA,
        ],
    )
    def gather_kernel(table_hbm, idx_hbm, out_hbm, idx_v, rows_v, sem):
        wid = lax.axis_index("s") * nc + lax.axis_index("c")
        base = wid * b_per_w
        pltpu.sync_copy(idx_hbm.at[pl.ds(base, b_per_w)], idx_v)
        pltpu.async_copy(table_hbm.at[idx_v], rows_v, sem).wait()
        pltpu.sync_copy(rows_v, out_hbm.at[pl.ds(base, b_per_w)])

    return gather_kernel(emb, zp)


def _banded_body(tlo_ref, tcnt_ref, nodef_ref, nodet_ref, xj_ref, iw1_ref,
                 ib1_ref, iw2_ref, ib2_ref, selr_ref, selc_ref, out_ref):
    b = pl.program_id(0)
    tlo = tlo_ref[b]
    tcnt = tcnt_ref[b]
    r0 = b * _R

    f32 = jnp.float32
    step = _CUTOFF / (_NG - 1)
    coeff = -0.5 / (step * step)
    gi = lax.broadcasted_iota(jnp.int32, (1, _NGP), 1).astype(f32)
    off = jnp.where(gi < _NG, gi * step, 1e6)

    rowdat = nodef_ref[pl.ds(r0, _R), :]
    br = rowdat[:, 0:1]
    nr = rowdat[:, 1:2]
    prx = rowdat[:, 2:3]
    pry = rowdat[:, 3:4]
    prz = rowdat[:, 4:5]
    ridx = r0 + lax.broadcasted_iota(jnp.int32, (_R, 1), 0)

    iw1v = iw1_ref[...]
    ib1v = ib1_ref[...]
    iw2v = iw2_ref[...]
    ib2v = ib2_ref[...]

    sel_r = selr_ref[...]
    sel_c = selc_ref[...]

    def _flatten2(a2, b2):
        ab = jnp.concatenate([a2, b2], axis=1)
        hi = ab.astype(jnp.bfloat16).astype(f32)
        lo = ab - hi
        ex = (jnp.dot(sel_r, hi, preferred_element_type=f32)
              + jnp.dot(sel_r, lo, preferred_element_type=f32))
        af = jnp.sum(ex[:, :_CT] * sel_c, axis=1, keepdims=True)
        bf = jnp.sum(ex[:, _CT:] * sel_c, axis=1, keepdims=True)
        return af, bf

    def tile_body(t, acc):
        tt = tlo + t
        c0 = tt * _CT
        colt = nodet_ref[pl.ds(tt, 1), :, :].reshape(8, _CT)
        xjc = xj_ref[pl.ds(c0, _CT), :]
        bc = colt[0:1, :]
        nc = colt[1:2, :]
        pcx = colt[2:3, :]
        pcy = colt[3:4, :]
        pcz = colt[4:5, :]
        cidx = c0 + lax.broadcasted_iota(jnp.int32, (1, _CT), 1)

        dot3 = prx * pcx + pry * pcy + prz * pcz
        d2 = (nr + nc) - 2.0 * dot3
        m = (d2 < _CUTOFF * _CUTOFF) & (br == bc) & (ridx != cidx)
        dx = prx - pcx
        dy = pry - pcy
        dz = prz - pcz
        ew = jnp.sqrt(dx * dx + dy * dy + dz * dz)
        cw = 0.5 * (jnp.cos(ew * (jnp.pi / _CUTOFF)) + 1.0)
        cm2 = jnp.where(m, cw, 0.0)

        ew_f, cm_f = _flatten2(ew, cm2)
        ea = jnp.exp(coeff * (ew_f - off) ** 2)
        t1 = _ssp(jnp.dot(ea, iw1v, preferred_element_type=f32) + ib1v)
        w = jnp.dot(t1, iw2v, preferred_element_type=f32) + ib2v
        v = w * cm_f
        v3 = v.reshape(_R, _CT, _HID)
        return acc + jnp.sum(v3 * xjc[None, :, :], axis=1)

    acc = lax.fori_loop(0, tcnt, tile_body, jnp.zeros((_R, _HID), f32))
    out_ref[...] = acc


def _banded(tlo, tcnt, nodef, nodet, xj, iw1p, ib1, iw2, ib2):
    grid_spec = pltpu.PrefetchScalarGridSpec(
        num_scalar_prefetch=2,
        grid=(_NB,),
        in_specs=[
            pl.BlockSpec((_NP, 8), lambda b, *_: (0, 0)),
            pl.BlockSpec((_NP // _CT, 8, _CT), lambda b, *_: (0, 0, 0)),
            pl.BlockSpec((_NP, _HID), lambda b, *_: (0, 0)),
            pl.BlockSpec((_NGP, _HID), lambda b, *_: (0, 0)),
            pl.BlockSpec((1, _HID), lambda b, *_: (0, 0)),
            pl.BlockSpec((_HID, _HID), lambda b, *_: (0, 0)),
            pl.BlockSpec((1, _HID), lambda b, *_: (0, 0)),
            pl.BlockSpec((_P, _R), lambda b, *_: (0, 0)),
            pl.BlockSpec((_P, _CT), lambda b, *_: (0, 0)),
        ],
        out_specs=pl.BlockSpec((_R, _HID), lambda b, *_: (b, 0)),
    )
    selr = (jnp.arange(_P, dtype=jnp.int32)[:, None] // _CT
            == jnp.arange(_R, dtype=jnp.int32)[None, :]).astype(jnp.float32)
    selc = (jnp.arange(_P, dtype=jnp.int32)[:, None] % _CT
            == jnp.arange(_CT, dtype=jnp.int32)[None, :]).astype(jnp.float32)
    return pl.pallas_call(
        _banded_body,
        grid_spec=grid_spec,
        out_shape=jax.ShapeDtypeStruct((_NP, _HID), jnp.float32),
    )(tlo, tcnt, nodef, nodet, xj, iw1p, ib1.reshape(1, _HID), iw2,
      ib2.reshape(1, _HID), selr, selc)


def _mm_body(x_ref, w_ref, o_ref):
    o_ref[...] = jnp.dot(x_ref[...], w_ref[...],
                         preferred_element_type=jnp.float32)


def _mm(x, w):
    m, k = x.shape
    n = w.shape[1]
    blk = min(1024, m)
    return pl.pallas_call(
        _mm_body,
        grid=(m // blk,),
        in_specs=[pl.BlockSpec((blk, k), lambda i: (i, 0)),
                  pl.BlockSpec((k, n), lambda i: (0, 0))],
        out_specs=pl.BlockSpec((blk, n), lambda i: (i, 0)),
        out_shape=jax.ShapeDtypeStruct((m, n), jnp.float32),
    )(x, w)


def _update_body(h_ref, agg_ref, cw2_ref, cb2_ref, lw_ref, lb_ref, cw1n_ref,
                 h_out, xj_out):
    t = _ssp(jnp.dot(agg_ref[...], cw2_ref[...],
                     preferred_element_type=jnp.float32) + cb2_ref[...])
    hn = h_ref[...] + jnp.dot(
        t, lw_ref[...], preferred_element_type=jnp.float32) + lb_ref[...]
    h_out[...] = hn
    xj_out[...] = jnp.dot(hn, cw1n_ref[...],
                          preferred_element_type=jnp.float32)


def _update(h, agg, cw2, cb2, lw, lb, cw1n):
    blk = min(1024, _NP)
    return pl.pallas_call(
        _update_body,
        grid=(_NP // blk,),
        in_specs=[pl.BlockSpec((blk, _HID), lambda i: (i, 0)),
                  pl.BlockSpec((blk, _HID), lambda i: (i, 0)),
                  pl.BlockSpec((_HID, _HID), lambda i: (0, 0)),
                  pl.BlockSpec((1, _HID), lambda i: (0, 0)),
                  pl.BlockSpec((_HID, _HID), lambda i: (0, 0)),
                  pl.BlockSpec((1, _HID), lambda i: (0, 0)),
                  pl.BlockSpec((_HID, _HID), lambda i: (0, 0))],
        out_specs=[pl.BlockSpec((blk, _HID), lambda i: (i, 0)),
                   pl.BlockSpec((blk, _HID), lambda i: (i, 0))],
        out_shape=[jax.ShapeDtypeStruct((_NP, _HID), jnp.float32),
                   jax.ShapeDtypeStruct((_NP, _HID), jnp.float32)],
    )(h, agg, cw2, cb2.reshape(1, _HID), lw, lb.reshape(1, _HID), cw1n)


def _lin1_body(h_ref, w_ref, b_ref, o_ref):
    o_ref[...] = jnp.dot(h_ref[...], w_ref[...],
                         preferred_element_type=jnp.float32) + b_ref[...]


def _lin1(h, w, b):
    blk = min(1024, _NP)
    n = w.shape[1]
    return pl.pallas_call(
        _lin1_body,
        grid=(_NP // blk,),
        in_specs=[pl.BlockSpec((blk, _HID), lambda i: (i, 0)),
                  pl.BlockSpec((_HID, n), lambda i: (0, 0)),
                  pl.BlockSpec((1, n), lambda i: (0, 0))],
        out_specs=pl.BlockSpec((blk, n), lambda i: (i, 0)),
        out_shape=jax.ShapeDtypeStruct((_NP, n), jnp.float32),
    )(h, w, b.reshape(1, n))


def _readout_body(p_ref, m1w_ref, m1b_ref, m2w_ref, m2b_ref, o_ref):
    t = jax.nn.relu(jnp.dot(p_ref[...], m1w_ref[...],
                            preferred_element_type=jnp.float32) + m1b_ref[...])
    o_ref[...] = jnp.dot(t, m2w_ref[...],
                         preferred_element_type=jnp.float32) + m2b_ref[...]


def _readout(pairp, m1w, m1b, m2w, m2b):
    mp = pairp.shape[0]
    blk = 512
    return pl.pallas_call(
        _readout_body,
        grid=(mp // blk,),
        in_specs=[pl.BlockSpec((blk, _HID), lambda i: (i, 0)),
                  pl.BlockSpec((_HID, _HID), lambda i: (0, 0)),
                  pl.BlockSpec((1, _HID), lambda i: (0, 0)),
                  pl.BlockSpec((_HID, 1), lambda i: (0, 0)),
                  pl.BlockSpec((1, 1), lambda i: (0, 0))],
        out_specs=pl.BlockSpec((blk, 1), lambda i: (i, 0)),
        out_shape=jax.ShapeDtypeStruct((mp, 1), jnp.float32),
    )(pairp, m1w, m1b.reshape(1, _HID), m2w, m2b.reshape(1, 1))


def kernel(z, batch, pos, edges, emb, iw1, ib1, iw2, ib2, cw1, cw2, cb2,
           lw, lb, lin1_w, lin1_b, m1w, m1b, m2w, m2b):
    n = pos.shape[0]
    flat = edges[0].reshape(-1)
    pos_s = jnp.take(pos, flat, axis=0).astype(jnp.float32)
    nrm = (pos_s * pos_s).sum(1)
    batch_i = batch.astype(jnp.int32)

    padn = _NP - n
    batch_f = jnp.pad(batch_i, (0, padn),
                      constant_values=2 ** 24 - 1).astype(jnp.float32)
    nrm_p = jnp.pad(nrm, (0, padn))
    pos_p = jnp.pad(pos_s, ((0, padn), (0, 0)))
    gidx = jnp.arange(_NP, dtype=jnp.float32)
    zeros = jnp.zeros((_NP,), jnp.float32)
    nodef = jnp.stack([batch_f, nrm_p, pos_p[:, 0], pos_p[:, 1],
                       pos_p[:, 2], gidx, zeros, zeros], axis=1)
    nodet = nodef.T.reshape(8, _NP // _CT, _CT).transpose(1, 0, 2)

    row0 = jnp.arange(_NB, dtype=jnp.int32) * _R
    rlast = jnp.minimum(row0 + _R - 1, n - 1)
    bfirst = batch_i[jnp.minimum(row0, n - 1)]
    cs = jnp.searchsorted(batch_i, bfirst, side="left").astype(jnp.int32)
    ce = jnp.searchsorted(batch_i, batch_i[rlast], side="right").astype(jnp.int32)
    tlo = cs // _CT
    thi = (ce + _CT - 1) // _CT
    tcnt = jnp.where(row0 < n, thi - tlo, 0).astype(jnp.int32)

    iw1p = jnp.pad(iw1, ((0, 0), (0, _NGP - _NG), (0, 0)))

    zp = jnp.pad(z.astype(jnp.int32), (0, padn))
    h = _sc_embed(emb.astype(jnp.float32), zp)

    xj = _mm(h, cw1[0])
    for i in range(6):
        agg = _banded(tlo, tcnt, nodef, nodet, xj, iw1p[i], ib1[i], iw2[i],
                      ib2[i])
        h, xj = _update(h, agg, cw2[i], cb2[i], lw[i], lb[i],
                        cw1[(i + 1) % 6])

    ne = _lin1(h, lin1_w, lin1_b)
    pair = ne[:n].reshape(n // 2, 2 * ne.shape[1])
    mp = 5120
    pairp = jnp.pad(pair, ((0, mp - n // 2), (0, 0)))
    outp = _readout(pairp, m1w, m1b, m2w, m2b)
    return outp[: n // 2, 0]
